# baseline (speedup 1.0000x reference)
import jax
import jax.numpy as jnp
from jax import lax
from jax.experimental import pallas as pl
from jax.experimental.pallas import tpu as pltpu
from jax.experimental.pallas import tpu_sc as plsc

_N = 4096
_ROWS = 2048
_NC, _NS, _L = 2, 16, 16
_NW = _NC * _NS
_RPW = _ROWS // _NW
_NCH = _N // _L
_U = 8
_LOW = 21
_NB = 1 << (32 - _LOW)
_MIN32 = -(2 ** 31)
_IMAX = 2 ** 31 - 1

_T1 = 82
_T2 = 4014
_F_LO = 0.02 * (_N - 1) - 81
_F_HI = 0.98 * (_N - 1) - 4013


def _sp(c, dtype=jnp.int32):
    return jnp.full((_L,), c, dtype)


def _biased_key_v(v):
    i = lax.bitcast_convert_type(v, jnp.int32)
    return i ^ ((i >> 31) | jnp.int32(_MIN32))


def _sc_body(x_hbm, o_hbm, x0, x1, o0, o1, hist, csum, ssum, buf,
             isem0, isem1, osem0, osem1):
    wid = lax.axis_index("s") * _NC + lax.axis_index("c")
    r_base = wid * _RPW
    iota = lax.iota(jnp.int32, _L)
    zero = _sp(0)
    one = _sp(1)
    imax = _sp(_IMAX)
    mnv = _sp(_MIN32)
    lowmask = _sp((1 << _LOW) - 1)

    @plsc.parallel_loop(0, _NB // _L, 1, unroll=_U)
    def _(j):
        hist[pl.ds(j * _L, _L)] = zero

    def compute_row(xb, ob, row, first):
        with jax.named_scope('ph_pa'):
            @plsc.parallel_loop(0, _NCH, 1, unroll=_U)
            def _(j):
                bk = _biased_key_v(xb[pl.ds(j * _L, _L)])
                bt = lax.shift_right_logical(bk, _sp(_LOW))
                plsc.addupdate_scatter(hist, [bt], one)

        iota16 = iota * 16
        with jax.named_scope('ph_p1'):
            @plsc.parallel_loop(0, 8, 1, unroll=2)
            def _(g2):
                acc = zero
                for l in range(16):
                    acc = acc + plsc.load_gather(hist, [g2 * 256 + iota16 + l])
                csum[pl.ds(g2 * 16, _L)] = acc

        ssum[...] = zero
        def p2(g, c):
            s = jnp.sum(csum[pl.ds(g * 16, _L)])
            plsc.store_scatter(ssum, [zero + g], zero + s, mask=iota == 0)
            return 0
        lax.fori_loop(0, 8, p2, 0)

        sv = ssum[...]
        cums_s = plsc.cumsum(sv)

        def locate(t):
            tv = _sp(t)
            g = jnp.min(jnp.where(cums_s >= tv, iota, _sp(99)))
            base_g = jnp.sum(jnp.where(iota < g, sv, zero))
            cv = csum[pl.ds(g * 16, _L)]
            cumc = plsc.cumsum(cv)
            jj = jnp.min(jnp.where(base_g + cumc >= tv, iota, _sp(99)))
            jch = g * 16 + jj
            base_j = base_g + jnp.sum(jnp.where(iota < jj, cv, zero))
            hv = hist[pl.ds(jch * 16, _L)]
            cumh = plsc.cumsum(hv)
            bb = jnp.min(jnp.where(base_j + cumh >= tv, iota, _sp(99)))
            bkt = jch * 16 + bb
            cb = base_j + jnp.sum(jnp.where(iota < bb, hv, zero))
            return bkt, cb

        with jax.named_scope('ph_locate'):
            b_lo, cb_lo = locate(_T1)
            b_hi, cb_hi = locate(_T2)

        with jax.named_scope('ph_cp'):
            @plsc.parallel_loop(0, _NCH, 1, unroll=_U, carry=zero)
            def off(j, off_c):
                bk = _biased_key_v(xb[pl.ds(j * _L, _L)])
                b = lax.shift_right_logical(bk, _sp(_LOW))
                mm = (b == b_lo) | (b == b_hi)
                cc = plsc.cumsum(jnp.where(mm, one, zero))
                plsc.store_scatter(buf, [off_c + cc - 1], bk, mask=mm)
                return off_c + plsc.all_reduce_population_count(mm)
        m_all = jnp.max(off)

        def bisect2(m, bkt1, cb1, bkt2, cb2):
            rv1 = _sp(_T1) - cb1
            rv2 = _sp(_T2) - cb2
            mv = zero + m
            bv1 = zero + bkt1
            bv2 = zero + bkt2

            def chunk_hits2(j, mid1, mid2, c1, c2):
                ch = buf[pl.ds(j * _L, _L)]
                top = lax.shift_right_logical(ch, _sp(_LOW))
                low = ch & lowmask
                val = (iota + j * _L) < mv
                hit1 = (top == bv1) & (low <= mid1) & val
                hit2 = (top == bv2) & (low <= mid2) & val
                c1 = c1 + jnp.where(hit1, one, zero)
                c2 = c2 + jnp.where(hit2, one, zero)
                return c1, c2

            def step(lo_v, hi_v, cnt_vec, rv):
                pred = (zero + jnp.sum(cnt_vec)) >= rv
                return (jnp.where(pred, lo_v, ((lo_v + hi_v) >> 1) + 1),
                        jnp.where(pred, (lo_v + hi_v) >> 1, hi_v))

            def outer_static(nstatic):
                def outer(it, c):
                    lo1, hi1, lo2, hi2 = c
                    mid1 = (lo1 + hi1) >> 1
                    mid2 = (lo2 + hi2) >> 1
                    c1 = zero
                    c2 = zero
                    for j in range(nstatic):
                        c1, c2 = chunk_hits2(j, mid1, mid2, c1, c2)
                    return step(lo1, hi1, c1, rv1) + step(lo2, hi2, c2, rv2)
                return outer

            def outer_slow(it, c):
                lo1, hi1, lo2, hi2 = c
                mid1 = (lo1 + hi1) >> 1
                mid2 = (lo2 + hi2) >> 1
                nch = (m + 15) // 16
                def inner(j, cc):
                    return chunk_hits2(j, mid1, mid2, cc[0], cc[1])
                c1, c2 = lax.fori_loop(0, nch, inner, (zero, zero))
                return step(lo1, hi1, c1, rv1) + step(lo2, hi2, c2, rv2)

            top21 = _sp((1 << _LOW) - 1)
            init = (zero, top21, zero, top21)
            lo1, _h1, lo2, _h2 = lax.cond(
                m <= 128,
                lambda: lax.fori_loop(0, _LOW, outer_static(8), init),
                lambda: lax.cond(
                    m <= 256,
                    lambda: lax.fori_loop(0, _LOW, outer_static(16), init),
                    lambda: lax.fori_loop(0, _LOW, outer_slow, init)))
            return (bkt1 << _LOW) | lo1, (bkt2 << _LOW) | lo2

        with jax.named_scope('ph_bisect'):
            bk_a, bk_b = bisect2(m_all, b_lo, cb_lo, b_hi, cb_hi)
        sk_a = bk_a ^ mnv
        sk_b = bk_b ^ mnv

        with jax.named_scope('ph_nb'):
            @plsc.parallel_loop(0, _NCH, 1, unroll=_U,
                                carry=(zero, imax, zero, imax))
            def nbc(j, carry):
                ca, ma, cbn, mb = carry
                sk = _biased_key_v(xb[pl.ds(j * _L, _L)]) ^ mnv
                lea = sk <= sk_a
                leb = sk <= sk_b
                ca = ca + jnp.where(lea, one, zero)
                cbn = cbn + jnp.where(leb, one, zero)
                ma = jnp.minimum(ma, jnp.where(lea, imax, sk))
                mb = jnp.minimum(mb, jnp.where(leb, imax, sk))
                @pl.when(j < _NB // _L)
                def _():
                    hist[pl.ds(j * _L, _L)] = zero
                return ca, ma, cbn, mb
        ca, ma, cbn, mb = nbc
        cnt_a = jnp.sum(ca)
        cnt_b = jnp.sum(cbn)
        gt_a = jnp.min(ma)
        gt_b = jnp.min(mb)
        sk_a1 = jnp.where(cnt_a >= _T1 + 1, sk_a, zero + gt_a)
        sk_b1 = jnp.where(cnt_b >= _T2 + 1, sk_b, zero + gt_b)

        def key_to_val(skv):
            iv = skv ^ ((skv >> 31) & _sp(0x7FFFFFFF))
            return lax.bitcast_convert_type(iv, jnp.float32)

        va = key_to_val(sk_a)
        va1 = key_to_val(sk_a1)
        vb = key_to_val(sk_b)
        vb1 = key_to_val(sk_b1)
        lower = va + jnp.float32(_F_LO) * (va1 - va)
        upper = vb + jnp.float32(_F_HI) * (vb1 - vb)
        inv = _sp(1.0, jnp.float32) / (upper - lower)

        @pl.when(jnp.logical_not(first))
        def _():
            pltpu.make_async_copy(ob, o_hbm.at[row], osem0 if ob is o0
                                  else osem1).wait()

        with jax.named_scope('ph_nm'):
            @plsc.parallel_loop(0, _NCH, 1, unroll=_U)
            def _(j):
                sl = pl.ds(j * _L, _L)
                ob[sl] = (xb[sl] - lower) * inv
        pltpu.make_async_copy(ob, o_hbm.at[row], osem0 if ob is o0
                              else osem1).start()

    pltpu.make_async_copy(x_hbm.at[r_base], x0, isem0).start()

    def pair_body(k, c):
        r0 = r_base + 2 * k
        r1 = r0 + 1
        pltpu.make_async_copy(x_hbm.at[r1], x1, isem1).start()
        pltpu.make_async_copy(x_hbm.at[r0], x0, isem0).wait()
        compute_row(x0, o0, r0, k == 0)

        @pl.when(k < _RPW // 2 - 1)
        def _():
            pltpu.make_async_copy(x_hbm.at[r0 + 2], x0, isem0).start()
        pltpu.make_async_copy(x_hbm.at[r1], x1, isem1).wait()
        compute_row(x1, o1, r1, k == 0)
        return 0

    lax.fori_loop(0, _RPW // 2, pair_body, 0)
    last = r_base + _RPW - 1
    pltpu.make_async_copy(o0, o_hbm.at[last - 1], osem0).wait()
    pltpu.make_async_copy(o1, o_hbm.at[last], osem1).wait()


@jax.jit
def kernel(x):
    b, c, n = x.shape
    xr = x.reshape(b * c, n)
    mesh = plsc.VectorSubcoreMesh(core_axis_name="c", subcore_axis_name="s",
                                  num_cores=_NC, num_subcores=_NS)
    fn = pl.kernel(
        _sc_body,
        out_type=jax.ShapeDtypeStruct((_ROWS, _N), jnp.float32),
        mesh=mesh,
        compiler_params=pltpu.CompilerParams(needs_layout_passes=False),
        scratch_types=[
            pltpu.VMEM((_N,), jnp.float32),
            pltpu.VMEM((_N,), jnp.float32),
            pltpu.VMEM((_N,), jnp.float32),
            pltpu.VMEM((_N,), jnp.float32),
            pltpu.VMEM((_NB,), jnp.int32),
            pltpu.VMEM((128,), jnp.int32),
            pltpu.VMEM((_L,), jnp.int32),
            pltpu.VMEM((_N,), jnp.int32),
            pltpu.SemaphoreType.DMA,
            pltpu.SemaphoreType.DMA,
            pltpu.SemaphoreType.DMA,
            pltpu.SemaphoreType.DMA,
        ],
    )
    return fn(xr).reshape(b, c, n)


# speedup vs baseline: 1.9644x; 1.9644x over previous
import jax
import jax.numpy as jnp
from jax import lax
from jax.experimental import pallas as pl
from jax.experimental.pallas import tpu as pltpu
from jax.experimental.pallas import tpu_sc as plsc

_N = 4096
_ROWS = 2048
_NC, _NS, _L = 2, 16, 16
_NW = _NC * _NS
_S_SC = 960
_RPW = _S_SC // _NW
_NCH = _N // _L
_U = 8
_LOW = 21
_NB = 1 << (32 - _LOW)
_MIN32 = -(2 ** 31)
_IMAX = 2 ** 31 - 1

_T1 = 82
_T2 = 4014
_F_LO = 0.02 * (_N - 1) - 81
_F_HI = 0.98 * (_N - 1) - 4013


def _sp(c, dtype=jnp.int32):
    return jnp.full((_L,), c, dtype)


def _biased_key_v(v):
    i = lax.bitcast_convert_type(v, jnp.int32)
    return i ^ ((i >> 31) | jnp.int32(_MIN32))


def _sc_body(x_hbm, o_hbm, x0, x1, o0, o1, hist, csum, ssum, buf,
             isem0, isem1, osem0, osem1):
    wid = lax.axis_index("s") * _NC + lax.axis_index("c")
    r_base = wid * _RPW
    iota = lax.iota(jnp.int32, _L)
    zero = _sp(0)
    one = _sp(1)
    imax = _sp(_IMAX)
    mnv = _sp(_MIN32)
    lowmask = _sp((1 << _LOW) - 1)

    @plsc.parallel_loop(0, _NB // _L, 1, unroll=_U)
    def _(j):
        hist[pl.ds(j * _L, _L)] = zero

    def compute_row(xb, ob, row, first):
        with jax.named_scope('ph_pa'):
            @plsc.parallel_loop(0, _NCH, 1, unroll=_U)
            def _(j):
                bk = _biased_key_v(xb[pl.ds(j * _L, _L)])
                bt = lax.shift_right_logical(bk, _sp(_LOW))
                plsc.addupdate_scatter(hist, [bt], one)

        iota16 = iota * 16
        with jax.named_scope('ph_p1'):
            @plsc.parallel_loop(0, 8, 1, unroll=2)
            def _(g2):
                acc = zero
                for l in range(16):
                    acc = acc + plsc.load_gather(hist, [g2 * 256 + iota16 + l])
                csum[pl.ds(g2 * 16, _L)] = acc

        ssum[...] = zero
        def p2(g, c):
            s = jnp.sum(csum[pl.ds(g * 16, _L)])
            plsc.store_scatter(ssum, [zero + g], zero + s, mask=iota == 0)
            return 0
        lax.fori_loop(0, 8, p2, 0)

        sv = ssum[...]
        cums_s = plsc.cumsum(sv)

        def locate(t):
            tv = _sp(t)
            g = jnp.min(jnp.where(cums_s >= tv, iota, _sp(99)))
            base_g = jnp.sum(jnp.where(iota < g, sv, zero))
            cv = csum[pl.ds(g * 16, _L)]
            cumc = plsc.cumsum(cv)
            jj = jnp.min(jnp.where(base_g + cumc >= tv, iota, _sp(99)))
            jch = g * 16 + jj
            base_j = base_g + jnp.sum(jnp.where(iota < jj, cv, zero))
            hv = hist[pl.ds(jch * 16, _L)]
            cumh = plsc.cumsum(hv)
            bb = jnp.min(jnp.where(base_j + cumh >= tv, iota, _sp(99)))
            bkt = jch * 16 + bb
            cb = base_j + jnp.sum(jnp.where(iota < bb, hv, zero))
            return bkt, cb

        with jax.named_scope('ph_locate'):
            b_lo, cb_lo = locate(_T1)
            b_hi, cb_hi = locate(_T2)

        with jax.named_scope('ph_cp'):
            @plsc.parallel_loop(0, _NCH, 1, unroll=_U, carry=zero)
            def off(j, off_c):
                bk = _biased_key_v(xb[pl.ds(j * _L, _L)])
                b = lax.shift_right_logical(bk, _sp(_LOW))
                mm = (b == b_lo) | (b == b_hi)
                cc = plsc.cumsum(jnp.where(mm, one, zero))
                plsc.store_scatter(buf, [off_c + cc - 1], bk, mask=mm)
                return off_c + plsc.all_reduce_population_count(mm)
        m_all = jnp.max(off)

        def bisect(m, bkt, cb, t):
            rv = _sp(t) - cb
            mv = zero + m
            bv = zero + bkt

            def step(lo_v, hi_v, cnt):
                pred = cnt >= rv
                return (jnp.where(pred, lo_v, ((lo_v + hi_v) >> 1) + 1),
                        jnp.where(pred, (lo_v + hi_v) >> 1, hi_v))

            def chunk_hits(j, mid):
                ch = buf[pl.ds(j * _L, _L)]
                inb = lax.shift_right_logical(ch, _sp(_LOW)) == bv
                hit = inb & ((ch & lowmask) <= mid) & ((iota + j * _L) < mv)
                return plsc.all_reduce_population_count(hit)

            def outer_static(nstatic):
                def outer(it, c):
                    lo_v, hi_v = c
                    mid = (lo_v + hi_v) >> 1
                    cnt = zero
                    for j in range(nstatic):
                        cnt = cnt + chunk_hits(j, mid)
                    return step(lo_v, hi_v, cnt)
                return outer

            def outer_slow(it, c):
                lo_v, hi_v = c
                mid = (lo_v + hi_v) >> 1
                nch = (m + 15) // 16
                def inner(j, acc):
                    return acc + chunk_hits(j, mid)
                cnt = lax.fori_loop(0, nch, inner, zero)
                return step(lo_v, hi_v, cnt)

            init = (zero, _sp((1 << _LOW) - 1))
            lo_v, _hi_v = lax.cond(
                m <= 128,
                lambda: lax.fori_loop(0, _LOW, outer_static(8), init),
                lambda: lax.cond(
                    m <= 256,
                    lambda: lax.fori_loop(0, _LOW, outer_static(16), init),
                    lambda: lax.fori_loop(0, _LOW, outer_slow, init)))
            return (bkt << _LOW) | lo_v

        with jax.named_scope('ph_bisect'):
            bk_a = bisect(m_all, b_lo, cb_lo, _T1)
            bk_b = bisect(m_all, b_hi, cb_hi, _T2)
        sk_a = bk_a ^ mnv
        sk_b = bk_b ^ mnv

        with jax.named_scope('ph_nb'):
            @plsc.parallel_loop(0, _NCH, 1, unroll=_U,
                                carry=(zero, imax, zero, imax))
            def nbc(j, carry):
                ca, ma, cbn, mb = carry
                sk = _biased_key_v(xb[pl.ds(j * _L, _L)]) ^ mnv
                lea = sk <= sk_a
                leb = sk <= sk_b
                ca = ca + jnp.where(lea, one, zero)
                cbn = cbn + jnp.where(leb, one, zero)
                ma = jnp.minimum(ma, jnp.where(lea, imax, sk))
                mb = jnp.minimum(mb, jnp.where(leb, imax, sk))
                @pl.when(j < _NB // _L)
                def _():
                    hist[pl.ds(j * _L, _L)] = zero
                return ca, ma, cbn, mb
        ca, ma, cbn, mb = nbc
        cnt_a = jnp.sum(ca)
        cnt_b = jnp.sum(cbn)
        gt_a = jnp.min(ma)
        gt_b = jnp.min(mb)
        sk_a1 = jnp.where(cnt_a >= _T1 + 1, sk_a, zero + gt_a)
        sk_b1 = jnp.where(cnt_b >= _T2 + 1, sk_b, zero + gt_b)

        def key_to_val(skv):
            iv = skv ^ ((skv >> 31) & _sp(0x7FFFFFFF))
            return lax.bitcast_convert_type(iv, jnp.float32)

        va = key_to_val(sk_a)
        va1 = key_to_val(sk_a1)
        vb = key_to_val(sk_b)
        vb1 = key_to_val(sk_b1)
        lower = va + jnp.float32(_F_LO) * (va1 - va)
        upper = vb + jnp.float32(_F_HI) * (vb1 - vb)
        inv = _sp(1.0, jnp.float32) / (upper - lower)

        @pl.when(jnp.logical_not(first))
        def _():
            pltpu.make_async_copy(ob, o_hbm.at[row], osem0 if ob is o0
                                  else osem1).wait()

        with jax.named_scope('ph_nm'):
            @plsc.parallel_loop(0, _NCH, 1, unroll=_U)
            def _(j):
                sl = pl.ds(j * _L, _L)
                ob[sl] = (xb[sl] - lower) * inv
        pltpu.make_async_copy(ob, o_hbm.at[row], osem0 if ob is o0
                              else osem1).start()

    pltpu.make_async_copy(x_hbm.at[r_base], x0, isem0).start()

    def pair_body(k, c):
        r0 = r_base + 2 * k
        r1 = r0 + 1
        pltpu.make_async_copy(x_hbm.at[r1], x1, isem1).start()
        pltpu.make_async_copy(x_hbm.at[r0], x0, isem0).wait()
        compute_row(x0, o0, r0, k == 0)

        @pl.when(k < _RPW // 2 - 1)
        def _():
            pltpu.make_async_copy(x_hbm.at[r0 + 2], x0, isem0).start()
        pltpu.make_async_copy(x_hbm.at[r1], x1, isem1).wait()
        compute_row(x1, o1, r1, k == 0)
        return 0

    lax.fori_loop(0, _RPW // 2, pair_body, 0)
    last = r_base + _RPW - 1
    pltpu.make_async_copy(o0, o_hbm.at[last - 1], osem0).wait()
    pltpu.make_async_copy(o1, o_hbm.at[last], osem1).wait()


_TBLK = 136


def _tc_to_key(x):
    i = jax.lax.bitcast_convert_type(x, jnp.int32)
    return i ^ ((i >> 31) & jnp.int32(0x7FFFFFFF))


def _tc_from_key(k):
    i = k ^ ((k >> 31) & jnp.int32(0x7FFFFFFF))
    return jax.lax.bitcast_convert_type(i, jnp.float32)


def _tc_kernel_body(x_ref, o_ref, key_ref, *, n, k_lo, k_hi, f_lo, f_hi):
    imin = -(2 ** 31)
    imax = 2 ** 31 - 1
    xb = x_ref[...]
    key_ref[...] = _tc_to_key(xb)

    r = xb.shape[0]
    lo1 = jnp.full((r, 1), imin, jnp.int32)
    hi1 = jnp.full((r, 1), imax, jnp.int32)
    lo2 = jnp.full((r, 1), imin, jnp.int32)
    hi2 = jnp.full((r, 1), imax, jnp.int32)

    def body(_, carry):
        lo1, hi1, lo2, hi2 = carry
        mid1 = (lo1 & hi1) + ((lo1 ^ hi1) >> 1)
        mid2 = (lo2 & hi2) + ((lo2 ^ hi2) >> 1)
        k = key_ref[...]
        c1 = jnp.sum((k <= mid1).astype(jnp.int32), axis=1, keepdims=True)
        c2 = jnp.sum((k <= mid2).astype(jnp.int32), axis=1, keepdims=True)
        p1 = c1 >= k_lo + 1
        p2 = c2 >= k_hi + 1
        return (jnp.where(p1, lo1, mid1 + 1), jnp.where(p1, mid1, hi1),
                jnp.where(p2, lo2, mid2 + 1), jnp.where(p2, mid2, hi2))

    lo1, hi1, lo2, hi2 = jax.lax.fori_loop(0, 32, body, (lo1, hi1, lo2, hi2))
    ka = lo1
    kb = lo2

    k = key_ref[...]
    le_a = k <= ka
    cnt_a = jnp.sum(le_a.astype(jnp.int32), axis=1, keepdims=True)
    min_gt_a = jnp.min(jnp.where(le_a, jnp.int32(imax), k), axis=1,
                       keepdims=True)
    ka1 = jnp.where(cnt_a >= k_lo + 2, ka, min_gt_a)

    le_b = k <= kb
    cnt_b = jnp.sum(le_b.astype(jnp.int32), axis=1, keepdims=True)
    min_gt_b = jnp.min(jnp.where(le_b, jnp.int32(imax), k), axis=1,
                       keepdims=True)
    kb1 = jnp.where(cnt_b >= k_hi + 2, kb, min_gt_b)

    va = _tc_from_key(ka)
    va1 = _tc_from_key(ka1)
    vb = _tc_from_key(kb)
    vb1 = _tc_from_key(kb1)
    lower = va + jnp.float32(f_lo) * (va1 - va)
    upper = vb + jnp.float32(f_hi) * (vb1 - vb)
    o_ref[...] = (xb - lower) / (upper - lower)


def tc_percentile_normalize(xr, *, n, k_lo, k_hi, f_lo, f_hi):
    import functools
    rows = xr.shape[0]
    body = functools.partial(_tc_kernel_body, n=n, k_lo=k_lo, k_hi=k_hi,
                             f_lo=f_lo, f_hi=f_hi)
    return pl.pallas_call(
        body,
        grid=(rows // _TBLK,),
        in_specs=[pl.BlockSpec((_TBLK, n), lambda i: (i, 0))],
        out_specs=pl.BlockSpec((_TBLK, n), lambda i: (i, 0)),
        out_shape=jax.ShapeDtypeStruct((rows, n), jnp.float32),
        scratch_shapes=[pltpu.VMEM((_TBLK, n), jnp.int32)],
    )(xr)


@jax.jit
def kernel(x):
    b, c, n = x.shape
    xr = x.reshape(b * c, n)
    mesh = plsc.VectorSubcoreMesh(core_axis_name="c", subcore_axis_name="s",
                                  num_cores=_NC, num_subcores=_NS)
    fn = pl.kernel(
        _sc_body,
        out_type=jax.ShapeDtypeStruct((_S_SC, _N), jnp.float32),
        mesh=mesh,
        compiler_params=pltpu.CompilerParams(needs_layout_passes=False),
        scratch_types=[
            pltpu.VMEM((_N,), jnp.float32),
            pltpu.VMEM((_N,), jnp.float32),
            pltpu.VMEM((_N,), jnp.float32),
            pltpu.VMEM((_N,), jnp.float32),
            pltpu.VMEM((_NB,), jnp.int32),
            pltpu.VMEM((128,), jnp.int32),
            pltpu.VMEM((_L,), jnp.int32),
            pltpu.VMEM((_N,), jnp.int32),
            pltpu.SemaphoreType.DMA,
            pltpu.SemaphoreType.DMA,
            pltpu.SemaphoreType.DMA,
            pltpu.SemaphoreType.DMA,
        ],
    )
    sc_out = fn(xr[:_S_SC])
    tc_out = tc_percentile_normalize(xr[_S_SC:], n=_N, k_lo=_T1 - 1,
                                     k_hi=_T2 - 1, f_lo=_F_LO, f_hi=_F_HI)
    return jnp.concatenate([sc_out, tc_out], axis=0).reshape(b, c, n)


# speedup vs baseline: 2.0406x; 1.0388x over previous
import jax
import jax.numpy as jnp
from jax import lax
from jax.experimental import pallas as pl
from jax.experimental.pallas import tpu as pltpu
from jax.experimental.pallas import tpu_sc as plsc

_N = 4096
_ROWS = 2048
_NC, _NS, _L = 2, 16, 16
_NW = _NC * _NS
_S_SC = 960
_RPW = _S_SC // _NW
_NCH = _N // _L
_U = 8
_LOW = 21
_NB = 1 << (32 - _LOW)
_MIN32 = -(2 ** 31)
_IMAX = 2 ** 31 - 1

_T1 = 82
_T2 = 4014
_F_LO = 0.02 * (_N - 1) - 81
_F_HI = 0.98 * (_N - 1) - 4013


def _sp(c, dtype=jnp.int32):
    return jnp.full((_L,), c, dtype)


def _biased_key_v(v):
    i = lax.bitcast_convert_type(v, jnp.int32)
    return i ^ ((i >> 31) | jnp.int32(_MIN32))


def _sc_body(x_hbm, o_hbm, x0, x1, o0, o1, hist, csum, ssum, buf,
             isem0, isem1, osem0, osem1):
    wid = lax.axis_index("s") * _NC + lax.axis_index("c")
    r_base = wid * _RPW
    iota = lax.iota(jnp.int32, _L)
    zero = _sp(0)
    one = _sp(1)
    imax = _sp(_IMAX)
    mnv = _sp(_MIN32)
    lowmask = _sp((1 << _LOW) - 1)

    @plsc.parallel_loop(0, _NB // _L, 1, unroll=_U)
    def _(j):
        hist[pl.ds(j * _L, _L)] = zero

    def compute_row(xb, ob, row, first):
        with jax.named_scope('ph_pa'):
            @plsc.parallel_loop(0, _NCH, 1, unroll=_U)
            def _(j):
                bk = _biased_key_v(xb[pl.ds(j * _L, _L)])
                bt = lax.shift_right_logical(bk, _sp(_LOW))
                plsc.addupdate_scatter(hist, [bt], one)

        iota16 = iota * 16
        with jax.named_scope('ph_p1'):
            @plsc.parallel_loop(0, 8, 1, unroll=2)
            def _(g2):
                acc = zero
                for l in range(16):
                    acc = acc + plsc.load_gather(hist, [g2 * 256 + iota16 + l])
                csum[pl.ds(g2 * 16, _L)] = acc

        ssum[...] = zero
        def p2(g, c):
            s = jnp.sum(csum[pl.ds(g * 16, _L)])
            plsc.store_scatter(ssum, [zero + g], zero + s, mask=iota == 0)
            return 0
        lax.fori_loop(0, 8, p2, 0)

        sv = ssum[...]
        cums_s = plsc.cumsum(sv)

        def locate(t):
            tv = _sp(t)
            g = jnp.min(jnp.where(cums_s >= tv, iota, _sp(99)))
            base_g = jnp.sum(jnp.where(iota < g, sv, zero))
            cv = csum[pl.ds(g * 16, _L)]
            cumc = plsc.cumsum(cv)
            jj = jnp.min(jnp.where(base_g + cumc >= tv, iota, _sp(99)))
            jch = g * 16 + jj
            base_j = base_g + jnp.sum(jnp.where(iota < jj, cv, zero))
            hv = hist[pl.ds(jch * 16, _L)]
            cumh = plsc.cumsum(hv)
            bb = jnp.min(jnp.where(base_j + cumh >= tv, iota, _sp(99)))
            bkt = jch * 16 + bb
            cb = base_j + jnp.sum(jnp.where(iota < bb, hv, zero))
            return bkt, cb

        with jax.named_scope('ph_locate'):
            b_lo, cb_lo = locate(_T1)
            b_hi, cb_hi = locate(_T2)

        with jax.named_scope('ph_cp'):
            @plsc.parallel_loop(0, _NCH, 1, unroll=_U, carry=zero)
            def off(j, off_c):
                bk = _biased_key_v(xb[pl.ds(j * _L, _L)])
                b = lax.shift_right_logical(bk, _sp(_LOW))
                mm = (b == b_lo) | (b == b_hi)
                cc = plsc.cumsum(jnp.where(mm, one, zero))
                plsc.store_scatter(buf, [off_c + cc - 1], bk, mask=mm)
                return off_c + plsc.all_reduce_population_count(mm)
        m_all = jnp.max(off)

        def bisect(m, bkt, cb, t):
            rv = _sp(t) - cb
            mv = zero + m
            bv = zero + bkt

            def step(lo_v, hi_v, cnt):
                pred = cnt >= rv
                return (jnp.where(pred, lo_v, ((lo_v + hi_v) >> 1) + 1),
                        jnp.where(pred, (lo_v + hi_v) >> 1, hi_v))

            def chunk_hits(j, mid):
                ch = buf[pl.ds(j * _L, _L)]
                inb = lax.shift_right_logical(ch, _sp(_LOW)) == bv
                hit = inb & ((ch & lowmask) <= mid) & ((iota + j * _L) < mv)
                return plsc.all_reduce_population_count(hit)

            def outer_static(nstatic):
                def outer(it, c):
                    lo_v, hi_v = c
                    mid = (lo_v + hi_v) >> 1
                    cnt = zero
                    for j in range(nstatic):
                        cnt = cnt + chunk_hits(j, mid)
                    return step(lo_v, hi_v, cnt)
                return outer

            def outer_slow(it, c):
                lo_v, hi_v = c
                mid = (lo_v + hi_v) >> 1
                nch = (m + 15) // 16
                def inner(j, acc):
                    return acc + chunk_hits(j, mid)
                cnt = lax.fori_loop(0, nch, inner, zero)
                return step(lo_v, hi_v, cnt)

            init = (zero, _sp((1 << _LOW) - 1))
            lo_v, _hi_v = lax.cond(
                m <= 128,
                lambda: lax.fori_loop(0, _LOW, outer_static(8), init),
                lambda: lax.cond(
                    m <= 256,
                    lambda: lax.fori_loop(0, _LOW, outer_static(16), init),
                    lambda: lax.fori_loop(0, _LOW, outer_slow, init)))
            return (bkt << _LOW) | lo_v

        with jax.named_scope('ph_bisect'):
            bk_a = bisect(m_all, b_lo, cb_lo, _T1)
            bk_b = bisect(m_all, b_hi, cb_hi, _T2)
        sk_a = bk_a ^ mnv
        sk_b = bk_b ^ mnv

        with jax.named_scope('ph_nb'):
            @plsc.parallel_loop(0, _NCH, 1, unroll=_U,
                                carry=(zero, imax, zero, imax))
            def nbc(j, carry):
                ca, ma, cbn, mb = carry
                sk = _biased_key_v(xb[pl.ds(j * _L, _L)]) ^ mnv
                lea = sk <= sk_a
                leb = sk <= sk_b
                ca = ca + jnp.where(lea, one, zero)
                cbn = cbn + jnp.where(leb, one, zero)
                ma = jnp.minimum(ma, jnp.where(lea, imax, sk))
                mb = jnp.minimum(mb, jnp.where(leb, imax, sk))
                @pl.when(j < _NB // _L)
                def _():
                    hist[pl.ds(j * _L, _L)] = zero
                return ca, ma, cbn, mb
        ca, ma, cbn, mb = nbc
        cnt_a = jnp.sum(ca)
        cnt_b = jnp.sum(cbn)
        gt_a = jnp.min(ma)
        gt_b = jnp.min(mb)
        sk_a1 = jnp.where(cnt_a >= _T1 + 1, sk_a, zero + gt_a)
        sk_b1 = jnp.where(cnt_b >= _T2 + 1, sk_b, zero + gt_b)

        def key_to_val(skv):
            iv = skv ^ ((skv >> 31) & _sp(0x7FFFFFFF))
            return lax.bitcast_convert_type(iv, jnp.float32)

        va = key_to_val(sk_a)
        va1 = key_to_val(sk_a1)
        vb = key_to_val(sk_b)
        vb1 = key_to_val(sk_b1)
        lower = va + jnp.float32(_F_LO) * (va1 - va)
        upper = vb + jnp.float32(_F_HI) * (vb1 - vb)
        inv = _sp(1.0, jnp.float32) / (upper - lower)

        @pl.when(jnp.logical_not(first))
        def _():
            pltpu.make_async_copy(ob, o_hbm.at[row], osem0 if ob is o0
                                  else osem1).wait()

        with jax.named_scope('ph_nm'):
            @plsc.parallel_loop(0, _NCH, 1, unroll=_U)
            def _(j):
                sl = pl.ds(j * _L, _L)
                ob[sl] = (xb[sl] - lower) * inv
        pltpu.make_async_copy(ob, o_hbm.at[row], osem0 if ob is o0
                              else osem1).start()

    pltpu.make_async_copy(x_hbm.at[r_base], x0, isem0).start()

    def pair_body(k, c):
        r0 = r_base + 2 * k
        r1 = r0 + 1
        pltpu.make_async_copy(x_hbm.at[r1], x1, isem1).start()
        pltpu.make_async_copy(x_hbm.at[r0], x0, isem0).wait()
        compute_row(x0, o0, r0, k == 0)

        @pl.when(k < _RPW // 2 - 1)
        def _():
            pltpu.make_async_copy(x_hbm.at[r0 + 2], x0, isem0).start()
        pltpu.make_async_copy(x_hbm.at[r1], x1, isem1).wait()
        compute_row(x1, o1, r1, k == 0)
        return 0

    lax.fori_loop(0, _RPW // 2, pair_body, 0)
    last = r_base + _RPW - 1
    pltpu.make_async_copy(o0, o_hbm.at[last - 1], osem0).wait()
    pltpu.make_async_copy(o1, o_hbm.at[last], osem1).wait()


_TBLK = 64


def _tc_to_key(x):
    i = jax.lax.bitcast_convert_type(x, jnp.int32)
    return i ^ ((i >> 31) & jnp.int32(0x7FFFFFFF))


def _tc_from_key(k):
    i = k ^ ((k >> 31) & jnp.int32(0x7FFFFFFF))
    return jax.lax.bitcast_convert_type(i, jnp.float32)


def _tc_kernel_body(x_ref, o_ref, key_ref, *, n, k_lo, k_hi, f_lo, f_hi):
    imin = -(2 ** 31)
    imax = 2 ** 31 - 1
    xb = x_ref[...]
    key_ref[...] = _tc_to_key(xb)

    r = xb.shape[0]
    lo1 = jnp.full((r, 1), imin, jnp.int32)
    hi1 = jnp.full((r, 1), imax, jnp.int32)
    lo2 = jnp.full((r, 1), imin, jnp.int32)
    hi2 = jnp.full((r, 1), imax, jnp.int32)

    def body(_, carry):
        lo1, hi1, lo2, hi2 = carry
        mid1 = (lo1 & hi1) + ((lo1 ^ hi1) >> 1)
        mid2 = (lo2 & hi2) + ((lo2 ^ hi2) >> 1)
        k = key_ref[...]
        c1 = jnp.sum((k <= mid1).astype(jnp.int32), axis=1, keepdims=True)
        c2 = jnp.sum((k <= mid2).astype(jnp.int32), axis=1, keepdims=True)
        p1 = c1 >= k_lo + 1
        p2 = c2 >= k_hi + 1
        return (jnp.where(p1, lo1, mid1 + 1), jnp.where(p1, mid1, hi1),
                jnp.where(p2, lo2, mid2 + 1), jnp.where(p2, mid2, hi2))

    lo1, hi1, lo2, hi2 = jax.lax.fori_loop(0, 32, body, (lo1, hi1, lo2, hi2))
    ka = lo1
    kb = lo2

    k = key_ref[...]
    le_a = k <= ka
    cnt_a = jnp.sum(le_a.astype(jnp.int32), axis=1, keepdims=True)
    min_gt_a = jnp.min(jnp.where(le_a, jnp.int32(imax), k), axis=1,
                       keepdims=True)
    ka1 = jnp.where(cnt_a >= k_lo + 2, ka, min_gt_a)

    le_b = k <= kb
    cnt_b = jnp.sum(le_b.astype(jnp.int32), axis=1, keepdims=True)
    min_gt_b = jnp.min(jnp.where(le_b, jnp.int32(imax), k), axis=1,
                       keepdims=True)
    kb1 = jnp.where(cnt_b >= k_hi + 2, kb, min_gt_b)

    va = _tc_from_key(ka)
    va1 = _tc_from_key(ka1)
    vb = _tc_from_key(kb)
    vb1 = _tc_from_key(kb1)
    lower = va + jnp.float32(f_lo) * (va1 - va)
    upper = vb + jnp.float32(f_hi) * (vb1 - vb)
    o_ref[...] = (xb - lower) / (upper - lower)


def tc_percentile_normalize(xr_full, *, n, rows_off, k_lo, k_hi, f_lo, f_hi):
    import functools
    rows = xr_full.shape[0] - rows_off
    off = rows_off // _TBLK
    body = functools.partial(_tc_kernel_body, n=n, k_lo=k_lo, k_hi=k_hi,
                             f_lo=f_lo, f_hi=f_hi)
    return pl.pallas_call(
        body,
        grid=(rows // _TBLK,),
        in_specs=[pl.BlockSpec((_TBLK, n), lambda i: (i + off, 0))],
        out_specs=pl.BlockSpec((_TBLK, n), lambda i: (i, 0)),
        out_shape=jax.ShapeDtypeStruct((rows, n), jnp.float32),
        scratch_shapes=[pltpu.VMEM((_TBLK, n), jnp.int32)],
    )(xr_full)


@jax.jit
def kernel(x):
    b, c, n = x.shape
    xr = x.reshape(b * c, n)
    mesh = plsc.VectorSubcoreMesh(core_axis_name="c", subcore_axis_name="s",
                                  num_cores=_NC, num_subcores=_NS)
    fn = pl.kernel(
        _sc_body,
        out_type=jax.ShapeDtypeStruct((_ROWS, _N), jnp.float32),
        mesh=mesh,
        compiler_params=pltpu.CompilerParams(needs_layout_passes=False),
        scratch_types=[
            pltpu.VMEM((_N,), jnp.float32),
            pltpu.VMEM((_N,), jnp.float32),
            pltpu.VMEM((_N,), jnp.float32),
            pltpu.VMEM((_N,), jnp.float32),
            pltpu.VMEM((_NB,), jnp.int32),
            pltpu.VMEM((128,), jnp.int32),
            pltpu.VMEM((_L,), jnp.int32),
            pltpu.VMEM((_N,), jnp.int32),
            pltpu.SemaphoreType.DMA,
            pltpu.SemaphoreType.DMA,
            pltpu.SemaphoreType.DMA,
            pltpu.SemaphoreType.DMA,
        ],
    )
    sc_out = fn(xr)
    tc_out = tc_percentile_normalize(xr, n=_N, rows_off=_S_SC, k_lo=_T1 - 1,
                                     k_hi=_T2 - 1, f_lo=_F_LO, f_hi=_F_HI)
    out = lax.dynamic_update_slice(sc_out, tc_out, (_S_SC, 0))
    return out.reshape(b, c, n)


# speedup vs baseline: 2.3860x; 1.1693x over previous
import jax
import jax.numpy as jnp
from jax import lax
from jax.experimental import pallas as pl
from jax.experimental.pallas import tpu as pltpu
from jax.experimental.pallas import tpu_sc as plsc

_N = 4096
_ROWS = 2048
_NC, _NS, _L = 2, 16, 16
_NW = _NC * _NS
_S_SC = 1024
_RPW = _S_SC // _NW
_NCH = _N // _L
_U = 8
_LOW = 21
_NB = 1 << (32 - _LOW)
_MIN32 = -(2 ** 31)
_IMAX = 2 ** 31 - 1

_T1 = 82
_T2 = 4014
_F_LO = 0.02 * (_N - 1) - 81
_F_HI = 0.98 * (_N - 1) - 4013


def _sp(c, dtype=jnp.int32):
    return jnp.full((_L,), c, dtype)


def _biased_key_v(v):
    i = lax.bitcast_convert_type(v, jnp.int32)
    return i ^ ((i >> 31) | jnp.int32(_MIN32))


def _sc_body(x_hbm, o_hbm, x0, x1, o0, o1, hist, csum, ssum, buf,
             isem0, isem1, osem0, osem1):
    wid = lax.axis_index("s") * _NC + lax.axis_index("c")
    r_base = wid * _RPW
    iota = lax.iota(jnp.int32, _L)
    zero = _sp(0)
    one = _sp(1)
    imax = _sp(_IMAX)
    mnv = _sp(_MIN32)
    lowmask = _sp((1 << _LOW) - 1)

    @plsc.parallel_loop(0, _NB // _L, 1, unroll=_U)
    def _(j):
        hist[pl.ds(j * _L, _L)] = zero

    def compute_row(xb, ob, row, first):
        with jax.named_scope('ph_pa'):
            @plsc.parallel_loop(0, _NCH, 1, unroll=_U)
            def _(j):
                bk = _biased_key_v(xb[pl.ds(j * _L, _L)])
                bt = lax.shift_right_logical(bk, _sp(_LOW))
                plsc.addupdate_scatter(hist, [bt], one)

        iota16 = iota * 16
        with jax.named_scope('ph_p1'):
            @plsc.parallel_loop(0, 8, 1, unroll=2)
            def _(g2):
                acc = zero
                for l in range(16):
                    acc = acc + plsc.load_gather(hist, [g2 * 256 + iota16 + l])
                csum[pl.ds(g2 * 16, _L)] = acc

        ssum[...] = zero
        def p2(g, c):
            s = jnp.sum(csum[pl.ds(g * 16, _L)])
            plsc.store_scatter(ssum, [zero + g], zero + s, mask=iota == 0)
            return 0
        lax.fori_loop(0, 8, p2, 0)

        sv = ssum[...]
        cums_s = plsc.cumsum(sv)

        def locate(t):
            tv = _sp(t)
            g = jnp.min(jnp.where(cums_s >= tv, iota, _sp(99)))
            base_g = jnp.sum(jnp.where(iota < g, sv, zero))
            cv = csum[pl.ds(g * 16, _L)]
            cumc = plsc.cumsum(cv)
            jj = jnp.min(jnp.where(base_g + cumc >= tv, iota, _sp(99)))
            jch = g * 16 + jj
            base_j = base_g + jnp.sum(jnp.where(iota < jj, cv, zero))
            hv = hist[pl.ds(jch * 16, _L)]
            cumh = plsc.cumsum(hv)
            bb = jnp.min(jnp.where(base_j + cumh >= tv, iota, _sp(99)))
            bkt = jch * 16 + bb
            cb = base_j + jnp.sum(jnp.where(iota < bb, hv, zero))
            return bkt, cb

        with jax.named_scope('ph_locate'):
            b_lo, cb_lo = locate(_T1)
            b_hi, cb_hi = locate(_T2)

        with jax.named_scope('ph_cp'):
            @plsc.parallel_loop(0, _NCH, 1, unroll=_U, carry=zero)
            def off(j, off_c):
                bk = _biased_key_v(xb[pl.ds(j * _L, _L)])
                b = lax.shift_right_logical(bk, _sp(_LOW))
                mm = (b == b_lo) | (b == b_hi)
                cc = plsc.cumsum(jnp.where(mm, one, zero))
                plsc.store_scatter(buf, [off_c + cc - 1], bk, mask=mm)
                return off_c + plsc.all_reduce_population_count(mm)
        m_all = jnp.max(off)

        def bisect(m, bkt, cb, t):
            rv = _sp(t) - cb
            mv = zero + m
            bv = zero + bkt

            def step(lo_v, hi_v, cnt):
                pred = cnt >= rv
                return (jnp.where(pred, lo_v, ((lo_v + hi_v) >> 1) + 1),
                        jnp.where(pred, (lo_v + hi_v) >> 1, hi_v))

            def chunk_hits(j, mid):
                ch = buf[pl.ds(j * _L, _L)]
                inb = lax.shift_right_logical(ch, _sp(_LOW)) == bv
                hit = inb & ((ch & lowmask) <= mid) & ((iota + j * _L) < mv)
                return plsc.all_reduce_population_count(hit)

            def outer_static(nstatic):
                def outer(it, c):
                    lo_v, hi_v = c
                    mid = (lo_v + hi_v) >> 1
                    cnt = zero
                    for j in range(nstatic):
                        cnt = cnt + chunk_hits(j, mid)
                    return step(lo_v, hi_v, cnt)
                return outer

            def outer_slow(it, c):
                lo_v, hi_v = c
                mid = (lo_v + hi_v) >> 1
                nch = (m + 15) // 16
                def inner(j, acc):
                    return acc + chunk_hits(j, mid)
                cnt = lax.fori_loop(0, nch, inner, zero)
                return step(lo_v, hi_v, cnt)

            init = (zero, _sp((1 << _LOW) - 1))
            lo_v, _hi_v = lax.cond(
                m <= 128,
                lambda: lax.fori_loop(0, _LOW, outer_static(8), init),
                lambda: lax.cond(
                    m <= 256,
                    lambda: lax.fori_loop(0, _LOW, outer_static(16), init),
                    lambda: lax.fori_loop(0, _LOW, outer_slow, init)))
            return (bkt << _LOW) | lo_v

        with jax.named_scope('ph_bisect'):
            bk_a = bisect(m_all, b_lo, cb_lo, _T1)
            bk_b = bisect(m_all, b_hi, cb_hi, _T2)
        sk_a = bk_a ^ mnv
        sk_b = bk_b ^ mnv

        with jax.named_scope('ph_nb'):
            @plsc.parallel_loop(0, _NCH, 1, unroll=_U,
                                carry=(zero, imax, zero, imax))
            def nbc(j, carry):
                ca, ma, cbn, mb = carry
                sk = _biased_key_v(xb[pl.ds(j * _L, _L)]) ^ mnv
                lea = sk <= sk_a
                leb = sk <= sk_b
                ca = ca + jnp.where(lea, one, zero)
                cbn = cbn + jnp.where(leb, one, zero)
                ma = jnp.minimum(ma, jnp.where(lea, imax, sk))
                mb = jnp.minimum(mb, jnp.where(leb, imax, sk))
                @pl.when(j < _NB // _L)
                def _():
                    hist[pl.ds(j * _L, _L)] = zero
                return ca, ma, cbn, mb
        ca, ma, cbn, mb = nbc
        cnt_a = jnp.sum(ca)
        cnt_b = jnp.sum(cbn)
        gt_a = jnp.min(ma)
        gt_b = jnp.min(mb)
        sk_a1 = jnp.where(cnt_a >= _T1 + 1, sk_a, zero + gt_a)
        sk_b1 = jnp.where(cnt_b >= _T2 + 1, sk_b, zero + gt_b)

        def key_to_val(skv):
            iv = skv ^ ((skv >> 31) & _sp(0x7FFFFFFF))
            return lax.bitcast_convert_type(iv, jnp.float32)

        va = key_to_val(sk_a)
        va1 = key_to_val(sk_a1)
        vb = key_to_val(sk_b)
        vb1 = key_to_val(sk_b1)
        lower = va + jnp.float32(_F_LO) * (va1 - va)
        upper = vb + jnp.float32(_F_HI) * (vb1 - vb)
        inv = _sp(1.0, jnp.float32) / (upper - lower)

        @pl.when(jnp.logical_not(first))
        def _():
            pltpu.make_async_copy(ob, o_hbm.at[row], osem0 if ob is o0
                                  else osem1).wait()

        with jax.named_scope('ph_nm'):
            @plsc.parallel_loop(0, _NCH, 1, unroll=_U)
            def _(j):
                sl = pl.ds(j * _L, _L)
                ob[sl] = (xb[sl] - lower) * inv
        pltpu.make_async_copy(ob, o_hbm.at[row], osem0 if ob is o0
                              else osem1).start()

    pltpu.make_async_copy(x_hbm.at[r_base], x0, isem0).start()

    def pair_body(k, c):
        r0 = r_base + 2 * k
        r1 = r0 + 1
        pltpu.make_async_copy(x_hbm.at[r1], x1, isem1).start()
        pltpu.make_async_copy(x_hbm.at[r0], x0, isem0).wait()
        compute_row(x0, o0, r0, k == 0)

        @pl.when(k < _RPW // 2 - 1)
        def _():
            pltpu.make_async_copy(x_hbm.at[r0 + 2], x0, isem0).start()
        pltpu.make_async_copy(x_hbm.at[r1], x1, isem1).wait()
        compute_row(x1, o1, r1, k == 0)
        return 0

    lax.fori_loop(0, _RPW // 2, pair_body, 0)
    last = r_base + _RPW - 1
    pltpu.make_async_copy(o0, o_hbm.at[last - 1], osem0).wait()
    pltpu.make_async_copy(o1, o_hbm.at[last], osem1).wait()


_TBLK = 256


def _tc_to_key(x):
    i = jax.lax.bitcast_convert_type(x, jnp.int32)
    return i ^ ((i >> 31) & jnp.int32(0x7FFFFFFF))


def _tc_from_key(k):
    i = k ^ ((k >> 31) & jnp.int32(0x7FFFFFFF))
    return jax.lax.bitcast_convert_type(i, jnp.float32)


def _tc_kernel_body(x_ref, o_ref, key_ref, *, n, k_lo, k_hi, f_lo, f_hi):
    imin = -(2 ** 31)
    imax = 2 ** 31 - 1
    xb = x_ref[...]
    key_ref[...] = _tc_to_key(xb)

    r = xb.shape[0]
    lo1 = jnp.full((r, 1), imin, jnp.int32)
    hi1 = jnp.full((r, 1), imax, jnp.int32)
    lo2 = jnp.full((r, 1), imin, jnp.int32)
    hi2 = jnp.full((r, 1), imax, jnp.int32)

    def body(_, carry):
        lo1, hi1, lo2, hi2 = carry
        mid1 = (lo1 & hi1) + ((lo1 ^ hi1) >> 1)
        mid2 = (lo2 & hi2) + ((lo2 ^ hi2) >> 1)
        k = key_ref[...]
        c1 = jnp.sum((k <= mid1).astype(jnp.int32), axis=1, keepdims=True)
        c2 = jnp.sum((k <= mid2).astype(jnp.int32), axis=1, keepdims=True)
        p1 = c1 >= k_lo + 1
        p2 = c2 >= k_hi + 1
        return (jnp.where(p1, lo1, mid1 + 1), jnp.where(p1, mid1, hi1),
                jnp.where(p2, lo2, mid2 + 1), jnp.where(p2, mid2, hi2))

    lo1, hi1, lo2, hi2 = jax.lax.fori_loop(0, 32, body, (lo1, hi1, lo2, hi2))
    ka = lo1
    kb = lo2

    k = key_ref[...]
    le_a = k <= ka
    cnt_a = jnp.sum(le_a.astype(jnp.int32), axis=1, keepdims=True)
    min_gt_a = jnp.min(jnp.where(le_a, jnp.int32(imax), k), axis=1,
                       keepdims=True)
    ka1 = jnp.where(cnt_a >= k_lo + 2, ka, min_gt_a)

    le_b = k <= kb
    cnt_b = jnp.sum(le_b.astype(jnp.int32), axis=1, keepdims=True)
    min_gt_b = jnp.min(jnp.where(le_b, jnp.int32(imax), k), axis=1,
                       keepdims=True)
    kb1 = jnp.where(cnt_b >= k_hi + 2, kb, min_gt_b)

    va = _tc_from_key(ka)
    va1 = _tc_from_key(ka1)
    vb = _tc_from_key(kb)
    vb1 = _tc_from_key(kb1)
    lower = va + jnp.float32(f_lo) * (va1 - va)
    upper = vb + jnp.float32(f_hi) * (vb1 - vb)
    o_ref[...] = (xb - lower) / (upper - lower)


def tc_percentile_normalize(xr_full, *, n, rows_off, k_lo, k_hi, f_lo, f_hi):
    import functools
    rows = xr_full.shape[0] - rows_off
    off = rows_off // _TBLK
    body = functools.partial(_tc_kernel_body, n=n, k_lo=k_lo, k_hi=k_hi,
                             f_lo=f_lo, f_hi=f_hi)
    return pl.pallas_call(
        body,
        grid=(rows // _TBLK,),
        in_specs=[pl.BlockSpec((_TBLK, n), lambda i: (i + off, 0))],
        out_specs=pl.BlockSpec((_TBLK, n), lambda i: (i, 0)),
        out_shape=jax.ShapeDtypeStruct((rows, n), jnp.float32),
        scratch_shapes=[pltpu.VMEM((_TBLK, n), jnp.int32)],
    )(xr_full)


@jax.jit
def kernel(x):
    b, c, n = x.shape
    xr = x.reshape(b * c, n)
    mesh = plsc.VectorSubcoreMesh(core_axis_name="c", subcore_axis_name="s",
                                  num_cores=_NC, num_subcores=_NS)
    fn = pl.kernel(
        _sc_body,
        out_type=jax.ShapeDtypeStruct((_ROWS, _N), jnp.float32),
        mesh=mesh,
        compiler_params=pltpu.CompilerParams(needs_layout_passes=False),
        scratch_types=[
            pltpu.VMEM((_N,), jnp.float32),
            pltpu.VMEM((_N,), jnp.float32),
            pltpu.VMEM((_N,), jnp.float32),
            pltpu.VMEM((_N,), jnp.float32),
            pltpu.VMEM((_NB,), jnp.int32),
            pltpu.VMEM((128,), jnp.int32),
            pltpu.VMEM((_L,), jnp.int32),
            pltpu.VMEM((_N,), jnp.int32),
            pltpu.SemaphoreType.DMA,
            pltpu.SemaphoreType.DMA,
            pltpu.SemaphoreType.DMA,
            pltpu.SemaphoreType.DMA,
        ],
    )
    sc_out = fn(xr)
    tc_out = tc_percentile_normalize(xr, n=_N, rows_off=_S_SC, k_lo=_T1 - 1,
                                     k_hi=_T2 - 1, f_lo=_F_LO, f_hi=_F_HI)
    out = lax.dynamic_update_slice(sc_out, tc_out, (_S_SC, 0))
    return out.reshape(b, c, n)
